# baseline (speedup 1.0000x reference)
import functools

import jax
import jax.numpy as jnp
from jax import lax
from jax.experimental import pallas as pl
from jax.experimental.pallas import tpu as pltpu
from jax.experimental.pallas import tpu_sc as plsc

C = 128
NBUF = 2


@functools.cache
def _make_gather(V2, B):
    info = plsc.get_sparse_core_info()
    NC, NS, L = info.num_cores, info.num_subcores, info.num_lanes
    NW = NC * NS
    assert B % NW == 0
    b_per_w = B // NW
    assert b_per_w % C == 0
    n_chunks = b_per_w // C
    assert n_chunks % NBUF == 0 and n_chunks >= 2 * NBUF
    mesh = plsc.VectorSubcoreMesh(core_axis_name="c", subcore_axis_name="s")

    @functools.partial(
        pl.kernel,
        mesh=mesh,
        out_type=jax.ShapeDtypeStruct((B, 2 * 64), jnp.float32),
        scratch_types=[
            pltpu.VMEM((b_per_w,), jnp.int32),
            pltpu.VMEM((b_per_w,), jnp.int32),
            [pltpu.VMEM((C, 2 * 64), jnp.float32)] * NBUF,
            [pltpu.SemaphoreType.DMA] * NBUF,
            [pltpu.SemaphoreType.DMA] * NBUF,
        ],
        compiler_params=pltpu.CompilerParams(
            use_tc_tiling_on_sc=True, needs_layout_passes=False
        ),
    )
    def gather_kernel(t2_hbm, idx_hbm, out_hbm, idx_v, half_v, rows, gsems, wsems):
        wid = lax.axis_index("s") * NC + lax.axis_index("c")
        base = wid * b_per_w
        pltpu.sync_copy(idx_hbm.at[pl.ds(base, b_per_w)], idx_v)

        @plsc.parallel_loop(0, b_per_w // L)
        def prep(g):
            ids = idx_v[pl.ds(g * L, L)]
            half_v[pl.ds(g * L, L)] = lax.shift_left(lax.bitwise_and(ids, 1), 6)
            idx_v[pl.ds(g * L, L)] = lax.shift_right_logical(ids, 1)

        def start_gather(c, b):
            pltpu.async_copy(
                t2_hbm.at[idx_v.at[pl.ds(c * C, C)]], rows[b], gsems[b]
            )

        def wait_gather(c, b):
            pltpu.make_async_copy(
                t2_hbm.at[idx_v.at[pl.ds(c * C, C)]], rows[b], gsems[b]
            ).wait()

        def select(c, b):
            def gbody(g, carry):
                hv = half_v[pl.ds(c * C + g * L, L)]
                for lp in range(L):
                    h = hv[lp]
                    row = g * L + lp
                    for k in range(4):
                        rows[b][row, pl.ds(k * L, L)] = (
                            rows[b][row, pl.ds(h + k * L, L)]
                        )
                return carry

            lax.fori_loop(0, C // L, gbody, 0)

        def start_write(c, b):
            pltpu.async_copy(
                rows[b], out_hbm.at[pl.ds(base + c * C, C)], wsems[b]
            )

        def wait_write(c, b):
            pltpu.make_async_copy(
                rows[b], out_hbm.at[pl.ds(base + c * C, C)], wsems[b]
            ).wait()

        def body(g, carry):
            for b in range(NBUF):
                c = g * NBUF + b

                @pl.when(c >= NBUF)
                def _():
                    wait_write(c - NBUF, b)

                start_gather(c, b)

                cw = c - (NBUF - 1)
                bw = (b - (NBUF - 1)) % NBUF

                @pl.when(cw >= 0)
                def _():
                    wait_gather(cw, bw)
                    select(cw, bw)
                    start_write(cw, bw)

            return carry

        lax.fori_loop(0, n_chunks // NBUF, body, 0)

        for j in range(NBUF - 1):
            cw = n_chunks - (NBUF - 1) + j
            bw = cw % NBUF
            wait_gather(cw, bw)
            select(cw, bw)
            start_write(cw, bw)
        for j in range(NBUF):
            c = n_chunks - NBUF + j
            wait_write(c, c % NBUF)

    return gather_kernel


def kernel(token_ids, table):
    V, D = table.shape
    B = token_ids.size
    t2 = table.reshape(V // 2, 2 * D)
    idx = token_ids.reshape(B).astype(jnp.int32)
    out128 = _make_gather(V // 2, B)(t2, idx)
    return out128[:, :D].reshape(*token_ids.shape, D)


# speedup vs baseline: 1.2755x; 1.2755x over previous
import functools

import jax
import jax.numpy as jnp
from jax import lax
from jax.experimental import pallas as pl
from jax.experimental.pallas import tpu as pltpu
from jax.experimental.pallas import tpu_sc as plsc

C = 256
NBUF = 2


@functools.cache
def _make_gather(V, B):
    info = plsc.get_sparse_core_info()
    NC, NS, L = info.num_cores, info.num_subcores, info.num_lanes
    NW = NC * NS
    assert B % NW == 0
    b_per_w = B // NW
    assert b_per_w % C == 0
    n_chunks = b_per_w // C
    assert n_chunks % NBUF == 0 and n_chunks >= 2 * NBUF
    mesh = plsc.VectorSubcoreMesh(core_axis_name="c", subcore_axis_name="s")

    @functools.partial(
        pl.kernel,
        mesh=mesh,
        out_type=jax.ShapeDtypeStruct((B, 128), jnp.float32),
        scratch_types=[
            pltpu.VMEM((b_per_w,), jnp.int32),
            [pltpu.VMEM((C, 128), jnp.float32)] * NBUF,
            [pltpu.SemaphoreType.DMA] * NBUF,
            [pltpu.SemaphoreType.DMA] * NBUF,
        ],
        compiler_params=pltpu.CompilerParams(
            use_tc_tiling_on_sc=True, needs_layout_passes=False
        ),
    )
    def gather_kernel(t3_hbm, idx_hbm, out_hbm, idx_v, rows, gsems, wsems):
        wid = lax.axis_index("s") * NC + lax.axis_index("c")
        base = wid * b_per_w
        pltpu.sync_copy(idx_hbm.at[pl.ds(base, b_per_w)], idx_v)

        def start_gather(c, b):
            pltpu.async_copy(
                t3_hbm.at[idx_v.at[pl.ds(c * C, C)]], rows[b], gsems[b]
            )

        def wait_gather(c, b):
            pltpu.make_async_copy(
                t3_hbm.at[idx_v.at[pl.ds(c * C, C)]], rows[b], gsems[b]
            ).wait()

        def start_write(c, b):
            pltpu.async_copy(
                rows[b], out_hbm.at[pl.ds(base + c * C, C)], wsems[b]
            )

        def wait_write(c, b):
            pltpu.make_async_copy(
                rows[b], out_hbm.at[pl.ds(base + c * C, C)], wsems[b]
            ).wait()

        def body(g, carry):
            for b in range(NBUF):
                c = g * NBUF + b

                @pl.when(c >= NBUF)
                def _():
                    wait_write(c - NBUF, b)

                start_gather(c, b)

                cw = c - (NBUF - 1)
                bw = (b - (NBUF - 1)) % NBUF

                @pl.when(cw >= 0)
                def _():
                    wait_gather(cw, bw)
                    start_write(cw, bw)

            return carry

        lax.fori_loop(0, n_chunks // NBUF, body, 0)

        for j in range(NBUF - 1):
            cw = n_chunks - (NBUF - 1) + j
            bw = cw % NBUF
            wait_gather(cw, bw)
            start_write(cw, bw)
        for j in range(NBUF):
            c = n_chunks - NBUF + j
            wait_write(c, c % NBUF)

    return gather_kernel


def kernel(token_ids, table):
    V, D = table.shape
    B = token_ids.size
    t3 = jnp.pad(table, ((0, 0), (0, 128 - D)))
    idx = token_ids.reshape(B).astype(jnp.int32)
    out128 = _make_gather(V, B)(t3, idx)
    return out128[:, :D].reshape(*token_ids.shape, D)


# speedup vs baseline: 1.2785x; 1.0023x over previous
import functools

import jax
import jax.numpy as jnp
from jax import lax
from jax.experimental import pallas as pl
from jax.experimental.pallas import tpu as pltpu
from jax.experimental.pallas import tpu_sc as plsc

C = 320
NBUF = 2


@functools.cache
def _make_gather(V, B):
    info = plsc.get_sparse_core_info()
    NC, NS, L = info.num_cores, info.num_subcores, info.num_lanes
    NW = NC * NS
    assert B % NW == 0
    b_per_w = B // NW
    assert b_per_w % C == 0
    n_chunks = b_per_w // C
    assert n_chunks % NBUF == 0 and n_chunks >= 2 * NBUF
    mesh = plsc.VectorSubcoreMesh(core_axis_name="c", subcore_axis_name="s")

    @functools.partial(
        pl.kernel,
        mesh=mesh,
        out_type=jax.ShapeDtypeStruct((B, 128), jnp.float32),
        scratch_types=[
            pltpu.VMEM((b_per_w,), jnp.int32),
            [pltpu.VMEM((C, 128), jnp.float32)] * NBUF,
            [pltpu.SemaphoreType.DMA] * NBUF,
            [pltpu.SemaphoreType.DMA] * NBUF,
        ],
        compiler_params=pltpu.CompilerParams(
            use_tc_tiling_on_sc=True, needs_layout_passes=False
        ),
    )
    def gather_kernel(t3_hbm, idx_hbm, out_hbm, idx_v, rows, gsems, wsems):
        wid = lax.axis_index("s") * NC + lax.axis_index("c")
        base = wid * b_per_w
        pltpu.sync_copy(idx_hbm.at[pl.ds(base, b_per_w)], idx_v)

        def start_gather(c, b):
            pltpu.async_copy(
                t3_hbm.at[idx_v.at[pl.ds(c * C, C)]], rows[b], gsems[b]
            )

        def wait_gather(c, b):
            pltpu.make_async_copy(
                t3_hbm.at[idx_v.at[pl.ds(c * C, C)]], rows[b], gsems[b]
            ).wait()

        def start_write(c, b):
            pltpu.async_copy(
                rows[b], out_hbm.at[pl.ds(base + c * C, C)], wsems[b]
            )

        def wait_write(c, b):
            pltpu.make_async_copy(
                rows[b], out_hbm.at[pl.ds(base + c * C, C)], wsems[b]
            ).wait()

        def body(g, carry):
            for b in range(NBUF):
                c = g * NBUF + b

                @pl.when(c >= NBUF)
                def _():
                    wait_write(c - NBUF, b)

                start_gather(c, b)

                cw = c - (NBUF - 1)
                bw = (b - (NBUF - 1)) % NBUF

                @pl.when(cw >= 0)
                def _():
                    wait_gather(cw, bw)
                    start_write(cw, bw)

            return carry

        lax.fori_loop(0, n_chunks // NBUF, body, 0)

        for j in range(NBUF - 1):
            cw = n_chunks - (NBUF - 1) + j
            bw = cw % NBUF
            wait_gather(cw, bw)
            start_write(cw, bw)
        for j in range(NBUF):
            c = n_chunks - NBUF + j
            wait_write(c, c % NBUF)

    return gather_kernel


def kernel(token_ids, table):
    V, D = table.shape
    B = token_ids.size
    t3 = jnp.pad(table, ((0, 0), (0, 128 - D)))
    idx = token_ids.reshape(B).astype(jnp.int32)
    out128 = _make_gather(V, B)(t3, idx)
    return out128[:, :D].reshape(*token_ids.shape, D)
